# baseline (speedup 1.0000x reference)
import jax
import jax.numpy as jnp
from jax import lax
from jax.experimental import pallas as pl
from jax.experimental.pallas import tpu as pltpu
from jax.experimental.pallas import tpu_sc as plsc

N_NODES = 100000
K = 32
D = 128
B = 16384

NC = 2
NS = 16
NW = NC * NS
BPW = B // NW
QB = 4
CK = QB * K
NCH = BPW // QB
LANES = 16
NV = D // LANES
INV_K = 1.0 / K
ISLC = 128


def _gcn_body(x_hbm, adj_hbm, table_hbm, out_hbm,
              x_v, estage, eidx, self_v, nb0, nb1, out0, out1,
              sem_s, sem_n0, sem_n1, sem_o0, sem_o1):
    wid = lax.axis_index("s") * NC + lax.axis_index("c")
    base = wid * BPW

    pltpu.sync_copy(x_hbm.at[pl.ds(base, BPW)], x_v)

    for j in range(BPW // ISLC):
        sl = pl.ds(j * ISLC, ISLC)
        pltpu.async_copy(table_hbm.at[x_v.at[sl]], self_v.at[sl], sem_s)

    for j in range(BPW // ISLC):
        sl = pl.ds(j * ISLC, ISLC)
        pltpu.sync_copy(adj_hbm.at[x_v.at[sl]], estage)

        def repack(lg, carry, j=j):
            gidx = (j * (ISLC // QB) + lg) * CK
            for q in range(QB):
                for h in range(K // LANES):
                    eidx[pl.ds(gidx + q * K + h * LANES, LANES)] = (
                        estage[lg * QB + q, pl.ds(h * LANES, LANES)])
            return carry

        lax.fori_loop(0, ISLC // QB, repack, 0)

    def fire_nb(g, nb, sem):
        pltpu.async_copy(table_hbm.at[eidx.at[pl.ds(g * CK, CK)]], nb, sem)

    def drain_nb(g, nb, sem):
        pltpu.make_async_copy(
            table_hbm.at[eidx.at[pl.ds(g * CK, CK)]], nb, sem).wait()

    def fire_out(g, out_v, sem):
        pltpu.async_copy(out_v, out_hbm.at[pl.ds(base + g * QB, QB)], sem)

    def drain_out(g, out_v, sem):
        pltpu.make_async_copy(
            out_v, out_hbm.at[pl.ds(base + g * QB, QB)], sem).wait()

    def compute(g, nb, out_v):
        for q in range(QB):
            accs = [nb[q * K, pl.ds(d * LANES, LANES)] for d in range(NV)]
            for k in range(1, K):
                for d in range(NV):
                    accs[d] = accs[d] + nb[q * K + k, pl.ds(d * LANES, LANES)]
            for d in range(NV):
                dsl = pl.ds(d * LANES, LANES)
                out_v[q, dsl] = accs[d] * INV_K + self_v[g * QB + q, dsl]

    fire_nb(0, nb0, sem_n0)
    fire_nb(1, nb1, sem_n1)
    for j in range(BPW // ISLC):
        sl = pl.ds(j * ISLC, ISLC)
        pltpu.make_async_copy(table_hbm.at[x_v.at[sl]], self_v.at[sl], sem_s).wait()

    bufs = ((nb0, sem_n0, out0, sem_o0), (nb1, sem_n1, out1, sem_o1))

    def step(i, carry):
        for b, (nb, semn, out_v, semo) in enumerate(bufs):
            g = 2 * i + b

            @pl.when(g >= 2)
            def _():
                drain_out(g - 2, out_v, semo)

            drain_nb(g, nb, semn)
            compute(g, nb, out_v)
            fire_out(g, out_v, semo)

            @pl.when(g + 2 < NCH)
            def _():
                fire_nb(g + 2, nb, semn)

        return carry

    lax.fori_loop(0, NCH // 2, step, 0)
    drain_out(NCH - 2, out0, sem_o0)
    drain_out(NCH - 1, out1, sem_o1)


def kernel(X, adj, table):
    x = jnp.reshape(X, (B,)).astype(jnp.int32)
    adj32 = adj.astype(jnp.int32)
    f = pl.kernel(
        _gcn_body,
        out_type=jax.ShapeDtypeStruct((B, D), jnp.float32),
        mesh=plsc.VectorSubcoreMesh(core_axis_name="c", subcore_axis_name="s"),
        compiler_params=pltpu.CompilerParams(use_tc_tiling_on_sc=False),
        scratch_types=[
            pltpu.VMEM((BPW,), jnp.int32),
            pltpu.VMEM((ISLC, K), jnp.int32),
            pltpu.VMEM((NCH * CK,), jnp.int32),
            pltpu.VMEM((BPW, D), jnp.float32),
            pltpu.VMEM((CK, D), jnp.float32),
            pltpu.VMEM((CK, D), jnp.float32),
            pltpu.VMEM((QB, D), jnp.float32),
            pltpu.VMEM((QB, D), jnp.float32),
            pltpu.SemaphoreType.DMA,
            pltpu.SemaphoreType.DMA,
            pltpu.SemaphoreType.DMA,
            pltpu.SemaphoreType.DMA,
            pltpu.SemaphoreType.DMA,
        ],
    )
    out = f(x, adj32, table)
    return jnp.reshape(out, (B, 1, D))
